# baseline (speedup 1.0000x reference)
import functools

import jax
import jax.numpy as jnp
from jax import lax
from jax.experimental import pallas as pl
from jax.experimental.pallas import tpu as pltpu
from jax.experimental.pallas import tpu_sc as plsc

H = 32
MAX_DIST = 5
VOCAB_E = 1537
VOCAB_E_PAD = 1544
SPATIAL_OFF = MAX_DIST * VOCAB_E_PAD
TABLE_ROWS = SPATIAL_OFF + 512


def _tables_body(e_ref, w_ref, s_ref, out_ref):
    e = e_ref[...] * (1.0 / 3.0)
    for d in range(MAX_DIST):
        out_ref[pl.ds(d * VOCAB_E_PAD, VOCAB_E_PAD), :] = jnp.dot(
            e, w_ref[d], preferred_element_type=jnp.float32)
    out_ref[pl.ds(SPATIAL_OFF, 512), :] = s_ref[...]


def _build_table(edge_enc_w, edge_dis_w, spatial_enc_w):
    e_pad = jnp.zeros((VOCAB_E_PAD, H), jnp.float32).at[:VOCAB_E].set(edge_enc_w)
    w = edge_dis_w.reshape(-1, H, H)[:MAX_DIST]
    return pl.pallas_call(
        _tables_body,
        out_shape=jax.ShapeDtypeStruct((TABLE_ROWS, H), jnp.float32),
    )(e_pad, w, spatial_enc_w)


def _sc_gather_fn(n_pairs):
    info = plsc.get_sparse_core_info()
    nw = info.num_cores * info.num_subcores
    per_w = n_pairs // nw
    chunks = per_w // 8
    mesh = plsc.VectorSubcoreMesh(core_axis_name="c", subcore_axis_name="s")

    @functools.partial(
        pl.kernel,
        mesh=mesh,
        out_type=jax.ShapeDtypeStruct((n_pairs, H), jnp.float32),
        scratch_types=[
            pltpu.VMEM((per_w * 15,), jnp.int32),
            pltpu.VMEM((per_w,), jnp.int32),
            pltpu.VMEM((4, 128), jnp.int32),
            pltpu.VMEM((4, 128, H), jnp.float32),
            pltpu.VMEM((4, 8, H), jnp.float32),
            pltpu.SemaphoreType.DMA,
            pltpu.SemaphoreType.DMA,
            pltpu.SemaphoreType.DMA,
            pltpu.SemaphoreType.DMA,
            pltpu.SemaphoreType.DMA,
            pltpu.SemaphoreType.DMA,
            pltpu.SemaphoreType.DMA,
            pltpu.SemaphoreType.DMA,
        ],
        compiler_params=pltpu.CompilerParams(
            use_tc_tiling_on_sc=False, needs_layout_passes=False),
    )
    def k(table_hbm, edge_hbm, spat_hbm, out_hbm,
          edge_v, spat_v, idx_v, rows_v, ob_v,
          g0, g1, g2, g3, o0, o1, o2, o3):
        gsems = (g0, g1, g2, g3)
        osems = (o0, o1, o2, o3)
        wid = lax.axis_index("s") * info.num_cores + lax.axis_index("c")
        b = wid >> 2
        mbase = (wid & 3) * per_w
        for q in range(15):
            pltpu.sync_copy(
                edge_hbm.at[pl.ds(b * (15 * 16384) + q * 16384 + mbase,
                                  per_w)],
                edge_v.at[pl.ds(q * per_w, per_w)])
        pltpu.sync_copy(spat_hbm.at[pl.ds(wid * per_w, per_w)], spat_v)
        lanes = lax.iota(jnp.int32, 16)
        offs = ((lanes * 11) >> 5) * VOCAB_E_PAD
        is15 = lanes == 15
        stride = lanes * per_w

        def build_fire(c, j):
            for r in range(8):
                p = jnp.minimum(c * 8 + r, per_w - 1)
                eidx = plsc.load_gather(
                    edge_v, [jnp.minimum(stride + p, 15 * per_w - 1)])
                spb = plsc.load_gather(
                    spat_v, [jnp.full((16,), 0, jnp.int32) + p])
                idx_v[j, pl.ds(16 * r, 16)] = jnp.where(is15, spb, eidx) + offs
            pltpu.async_copy(table_hbm.at[idx_v.at[j]], rows_v.at[j],
                             gsems[j])

        def compute(c, j):
            for r in range(8):
                p = c * 8 + r
                acc_lo = rows_v[j, 16 * r, pl.ds(0, 16)]
                acc_hi = rows_v[j, 16 * r, pl.ds(16, 16)]
                for q in range(1, 15):
                    acc_lo = acc_lo + rows_v[j, 16 * r + q, pl.ds(0, 16)]
                    acc_hi = acc_hi + rows_v[j, 16 * r + q, pl.ds(16, 16)]
                s = plsc.load_gather(
                    spat_v, [jnp.full((16,), 0, jnp.int32) + p])
                s = jnp.where(s == 0, 1, s)
                s = jnp.where(s > 1, s - 1, s)
                s = jnp.minimum(s, MAX_DIST).astype(jnp.float32)
                ob_v[j, r, pl.ds(0, 16)] = (
                    acc_lo / s + rows_v[j, 16 * r + 15, pl.ds(0, 16)])
                ob_v[j, r, pl.ds(16, 16)] = (
                    acc_hi / s + rows_v[j, 16 * r + 15, pl.ds(16, 16)])
            pltpu.sync_copy(ob_v.at[j],
                            out_hbm.at[pl.ds(wid * per_w + 8 * c, 8)])

        def wait_gather(j):
            pltpu.make_async_copy(table_hbm.at[idx_v.at[j]], rows_v.at[j],
                                  gsems[j]).wait()

        for j in range(3):
            build_fire(jnp.int32(j), j)

        def group_body(g, carry):
            for j in range(4):
                c = g * 4 + j
                build_fire(c + 3, (j + 3) % 4)
                wait_gather(j)
                compute(c, j)
            return carry

        lax.fori_loop(0, chunks // 4, group_body, 0)
        for j in range(3):
            wait_gather(j)

    return k


def _assemble_body(ab_ref, tmp_ref, virt_ref, out_ref):
    ab2 = ab_ref[0] * 2.0
    tt = jnp.transpose(tmp_ref[0], (1, 0)).reshape(H, 128, 128)
    vw = virt_ref[...].reshape(H, 1, 1)
    out_ref[0, :, 0:1, :] = ab2[None, 0:1, :] + vw
    out_ref[0, :, 1:, 0:1] = ab2[None, 1:, 0:1] + vw
    out_ref[0, :, 1:, 1:] = tt + ab2[None, 1:, 1:]


def _assemble(attn_bias, tmp, virt_w, B, N):
    return pl.pallas_call(
        _assemble_body,
        grid=(B,),
        in_specs=[
            pl.BlockSpec((1, N + 1, N + 1), lambda i: (i, 0, 0)),
            pl.BlockSpec((1, N * N, H), lambda i: (i, 0, 0)),
            pl.BlockSpec((1, H), lambda i: (0, 0)),
        ],
        out_specs=pl.BlockSpec((1, H, N + 1, N + 1), lambda i: (i, 0, 0, 0)),
        out_shape=jax.ShapeDtypeStruct((B, H, N + 1, N + 1), jnp.float32),
    )(attn_bias, tmp.reshape(B, N * N, H), virt_w)


def kernel(attn_bias, spatial_pos, x, edge_input, attn_edge_type,
           edge_enc_w, edge_dis_w, spatial_enc_w, virt_w):
    B, N = x.shape[0], x.shape[1]
    n_pairs = B * N * N
    table = _build_table(edge_enc_w, edge_dis_w, spatial_enc_w)
    edge_flat = jnp.transpose(edge_input[..., :MAX_DIST, :],
                              (0, 3, 4, 1, 2)).reshape(-1).astype(jnp.int32)
    spat_flat = spatial_pos.reshape(-1).astype(jnp.int32)
    tmp = _sc_gather_fn(n_pairs)(table, edge_flat, spat_flat)
    return _assemble(attn_bias, tmp, virt_w, B, N)


# speedup vs baseline: 1.0042x; 1.0042x over previous
import functools

import jax
import jax.numpy as jnp
from jax import lax
from jax.experimental import pallas as pl
from jax.experimental.pallas import tpu as pltpu
from jax.experimental.pallas import tpu_sc as plsc

H = 32
MAX_DIST = 5
VOCAB_E = 1537
VOCAB_E_PAD = 1544
SPATIAL_OFF = MAX_DIST * VOCAB_E_PAD
TABLE_ROWS = SPATIAL_OFF + 512


def _tables_body(e_ref, w_ref, s_ref, out_ref):
    e = e_ref[...] * (1.0 / 3.0)
    for d in range(MAX_DIST):
        out_ref[pl.ds(d * VOCAB_E_PAD, VOCAB_E_PAD), :] = jnp.dot(
            e, w_ref[d], preferred_element_type=jnp.float32)
    out_ref[pl.ds(SPATIAL_OFF, 512), :] = s_ref[...]


def _build_table(edge_enc_w, edge_dis_w, spatial_enc_w):
    e_pad = jnp.zeros((VOCAB_E_PAD, H), jnp.float32).at[:VOCAB_E].set(edge_enc_w)
    w = edge_dis_w.reshape(-1, H, H)[:MAX_DIST]
    return pl.pallas_call(
        _tables_body,
        out_shape=jax.ShapeDtypeStruct((TABLE_ROWS, H), jnp.float32),
    )(e_pad, w, spatial_enc_w)


def _sc_gather_fn(n_pairs):
    info = plsc.get_sparse_core_info()
    nw = info.num_cores * info.num_subcores
    per_w = n_pairs // nw
    chunks = per_w // 8
    mesh = plsc.VectorSubcoreMesh(core_axis_name="c", subcore_axis_name="s")

    @functools.partial(
        pl.kernel,
        mesh=mesh,
        out_type=jax.ShapeDtypeStruct((n_pairs, H), jnp.float32),
        scratch_types=[
            pltpu.VMEM((16 + per_w * 15,), jnp.int32),
            pltpu.VMEM((16 + per_w,), jnp.int32),
            pltpu.VMEM((4, 128), jnp.int32),
            pltpu.VMEM((4, 128, H), jnp.float32),
            pltpu.VMEM((4, 8, H), jnp.float32),
            pltpu.SemaphoreType.DMA,
            pltpu.SemaphoreType.DMA,
            pltpu.SemaphoreType.DMA,
            pltpu.SemaphoreType.DMA,
            pltpu.SemaphoreType.DMA,
            pltpu.SemaphoreType.DMA,
            pltpu.SemaphoreType.DMA,
            pltpu.SemaphoreType.DMA,
        ],
        compiler_params=pltpu.CompilerParams(
            use_tc_tiling_on_sc=False, needs_layout_passes=False),
    )
    def k(table_hbm, edge_hbm, spat_hbm, out_hbm,
          edge_v, spat_v, idx_v, rows_v, ob_v,
          g0, g1, g2, g3, o0, o1, o2, o3):
        gsems = (g0, g1, g2, g3)
        osems = (o0, o1, o2, o3)
        wid = lax.axis_index("s") * info.num_cores + lax.axis_index("c")
        b = wid >> 2
        mbase = (wid & 3) * per_w
        for q in range(15):
            pltpu.sync_copy(
                edge_hbm.at[pl.ds(b * (15 * 16384) + q * 16384 + mbase,
                                  per_w)],
                edge_v.at[pl.ds(16 + q * per_w, per_w)])
        pltpu.sync_copy(spat_hbm.at[pl.ds(wid * per_w, per_w)],
                        spat_v.at[pl.ds(16, per_w)])
        lanes = lax.iota(jnp.int32, 16)
        offs = ((lanes * 11) >> 5) * VOCAB_E_PAD
        is15 = lanes == 15
        stride = lanes * per_w

        def build_fire(c, j):
            for r in range(8):
                p = jnp.minimum(c * 8 + r, per_w - 1)
                eidx = plsc.load_gather(
                    edge_v, [jnp.minimum(16 + stride + p, 15 + 15 * per_w)])
                spb = plsc.load_gather(
                    spat_v, [jnp.full((16,), 16, jnp.int32) + p])
                idx_v[j, pl.ds(16 * r, 16)] = jnp.where(is15, spb, eidx) + offs
            pltpu.async_copy(table_hbm.at[idx_v.at[j]], rows_v.at[j],
                             gsems[j])

        def compute(c, j):
            for r in range(8):
                p = c * 8 + r
                acc_lo = rows_v[j, 16 * r, pl.ds(0, 16)]
                acc_hi = rows_v[j, 16 * r, pl.ds(16, 16)]
                for q in range(1, 15):
                    acc_lo = acc_lo + rows_v[j, 16 * r + q, pl.ds(0, 16)]
                    acc_hi = acc_hi + rows_v[j, 16 * r + q, pl.ds(16, 16)]
                s = plsc.load_gather(
                    spat_v, [jnp.full((16,), 16, jnp.int32) + p])
                s = jnp.where(s == 0, 1, s)
                s = jnp.where(s > 1, s - 1, s)
                s = jnp.minimum(s, MAX_DIST).astype(jnp.float32)
                ob_v[j, r, pl.ds(0, 16)] = (
                    acc_lo / s + rows_v[j, 16 * r + 15, pl.ds(0, 16)])
                ob_v[j, r, pl.ds(16, 16)] = (
                    acc_hi / s + rows_v[j, 16 * r + 15, pl.ds(16, 16)])
            pltpu.sync_copy(ob_v.at[j],
                            out_hbm.at[pl.ds(wid * per_w + 8 * c, 8)])

        def wait_gather(j):
            pltpu.make_async_copy(table_hbm.at[idx_v.at[j]], rows_v.at[j],
                                  gsems[j]).wait()

        czero = jnp.minimum(wid * 0, 0)
        for j in range(3):
            build_fire(czero + j, j)

        def group_body(g, carry):
            for j in range(4):
                c = g * 4 + j
                build_fire(c + 3, (j + 3) % 4)
                wait_gather(j)
                compute(c, j)
            return carry

        lax.fori_loop(0, chunks // 4, group_body, 0)
        for j in range(3):
            wait_gather(j)

    return k


def _assemble_body(ab_ref, tmp_ref, virt_ref, out_ref):
    ab2 = ab_ref[0] * 2.0
    tt = jnp.transpose(tmp_ref[0], (1, 0)).reshape(H, 128, 128)
    vw = virt_ref[...].reshape(H, 1, 1)
    out_ref[0, :, 0:1, :] = ab2[None, 0:1, :] + vw
    out_ref[0, :, 1:, 0:1] = ab2[None, 1:, 0:1] + vw
    out_ref[0, :, 1:, 1:] = tt + ab2[None, 1:, 1:]


def _assemble(attn_bias, tmp, virt_w, B, N):
    return pl.pallas_call(
        _assemble_body,
        grid=(B,),
        in_specs=[
            pl.BlockSpec((1, N + 1, N + 1), lambda i: (i, 0, 0)),
            pl.BlockSpec((1, N * N, H), lambda i: (i, 0, 0)),
            pl.BlockSpec((1, H), lambda i: (0, 0)),
        ],
        out_specs=pl.BlockSpec((1, H, N + 1, N + 1), lambda i: (i, 0, 0, 0)),
        out_shape=jax.ShapeDtypeStruct((B, H, N + 1, N + 1), jnp.float32),
    )(attn_bias, tmp.reshape(B, N * N, H), virt_w)


def kernel(attn_bias, spatial_pos, x, edge_input, attn_edge_type,
           edge_enc_w, edge_dis_w, spatial_enc_w, virt_w):
    B, N = x.shape[0], x.shape[1]
    n_pairs = B * N * N
    table = _build_table(edge_enc_w, edge_dis_w, spatial_enc_w)
    edge_flat = jnp.transpose(edge_input[..., :MAX_DIST, :],
                              (0, 3, 4, 1, 2)).reshape(-1).astype(jnp.int32)
    spat_flat = spatial_pos.reshape(-1).astype(jnp.int32)
    tmp = _sc_gather_fn(n_pairs)(table, edge_flat, spat_flat)
    return _assemble(attn_bias, tmp, virt_w, B, N)


# speedup vs baseline: 1.0559x; 1.0515x over previous
import functools

import jax
import jax.numpy as jnp
from jax import lax
from jax.experimental import pallas as pl
from jax.experimental.pallas import tpu as pltpu
from jax.experimental.pallas import tpu_sc as plsc

H = 32
MAX_DIST = 5
VOCAB_E = 1537
VOCAB_E_PAD = 1544
SPATIAL_OFF = MAX_DIST * VOCAB_E_PAD
TABLE_ROWS = SPATIAL_OFF + 512


def _tables_body(e_ref, w_ref, s_ref, out_ref):
    e = e_ref[...] * (1.0 / 3.0)
    for d in range(MAX_DIST):
        out_ref[pl.ds(d * VOCAB_E_PAD, VOCAB_E_PAD), :] = jnp.dot(
            e, w_ref[d], preferred_element_type=jnp.float32)
    out_ref[pl.ds(SPATIAL_OFF, 512), :] = s_ref[...]


def _build_table(edge_enc_w, edge_dis_w, spatial_enc_w):
    e_pad = jnp.zeros((VOCAB_E_PAD, H), jnp.float32).at[:VOCAB_E].set(edge_enc_w)
    w = edge_dis_w.reshape(-1, H, H)[:MAX_DIST]
    return pl.pallas_call(
        _tables_body,
        out_shape=jax.ShapeDtypeStruct((TABLE_ROWS, H), jnp.float32),
    )(e_pad, w, spatial_enc_w)


def _sc_gather_fn(n_pairs):
    info = plsc.get_sparse_core_info()
    nw = info.num_cores * info.num_subcores
    per_w = n_pairs // nw
    chunks = per_w // 8
    mesh = plsc.VectorSubcoreMesh(core_axis_name="c", subcore_axis_name="s")

    @functools.partial(
        pl.kernel,
        mesh=mesh,
        out_type=jax.ShapeDtypeStruct((n_pairs, H), jnp.float32),
        scratch_types=[
            pltpu.VMEM((16 + per_w * 15,), jnp.int32),
            pltpu.VMEM((16 + per_w,), jnp.int32),
            pltpu.VMEM((4, 128), jnp.int32),
            pltpu.VMEM((4, 128, H), jnp.float32),
            pltpu.VMEM((4, 8, H), jnp.float32),
            pltpu.SemaphoreType.DMA,
            pltpu.SemaphoreType.DMA,
            pltpu.SemaphoreType.DMA,
            pltpu.SemaphoreType.DMA,
            pltpu.SemaphoreType.DMA,
            pltpu.SemaphoreType.DMA,
            pltpu.SemaphoreType.DMA,
            pltpu.SemaphoreType.DMA,
        ],
        compiler_params=pltpu.CompilerParams(
            use_tc_tiling_on_sc=False, needs_layout_passes=False),
    )
    def k(table_hbm, edge_hbm, spat_hbm, out_hbm,
          edge_v, spat_v, idx_v, rows_v, ob_v,
          g0, g1, g2, g3, o0, o1, o2, o3):
        gsems = (g0, g1, g2, g3)
        osems = (o0, o1, o2, o3)
        wid = lax.axis_index("s") * info.num_cores + lax.axis_index("c")
        b = wid >> 2
        mbase = (wid & 3) * per_w
        for q in range(15):
            pltpu.sync_copy(
                edge_hbm.at[pl.ds(b * (15 * 16384) + q * 16384 + mbase,
                                  per_w)],
                edge_v.at[pl.ds(16 + q * per_w, per_w)])
        pltpu.sync_copy(spat_hbm.at[pl.ds(wid * per_w, per_w)],
                        spat_v.at[pl.ds(16, per_w)])
        lanes = lax.iota(jnp.int32, 16)
        offs = ((lanes * 11) >> 5) * VOCAB_E_PAD
        is15 = lanes == 15
        stride = lanes * per_w

        def build_fire(c, j):
            for r in range(8):
                p = jnp.minimum(c * 8 + r, per_w - 1)
                eidx = plsc.load_gather(
                    edge_v, [jnp.minimum(16 + stride + p, 15 + 15 * per_w)])
                spb = plsc.load_gather(
                    spat_v, [jnp.full((16,), 16, jnp.int32) + p])
                idx_v[j, pl.ds(16 * r, 16)] = jnp.where(is15, spb, eidx) + offs
            pltpu.async_copy(table_hbm.at[idx_v.at[j]], rows_v.at[j],
                             gsems[j])

        def compute(c, j):
            for r in range(8):
                p = c * 8 + r
                acc_lo = rows_v[j, 16 * r, pl.ds(0, 16)]
                acc_hi = rows_v[j, 16 * r, pl.ds(16, 16)]
                for q in range(1, 15):
                    acc_lo = acc_lo + rows_v[j, 16 * r + q, pl.ds(0, 16)]
                    acc_hi = acc_hi + rows_v[j, 16 * r + q, pl.ds(16, 16)]
                s = plsc.load_gather(
                    spat_v, [jnp.full((16,), 16, jnp.int32) + p])
                s = jnp.where(s == 0, 1, s)
                s = jnp.where(s > 1, s - 1, s)
                s = jnp.minimum(s, MAX_DIST).astype(jnp.float32)
                ob_v[j, r, pl.ds(0, 16)] = (
                    acc_lo / s + rows_v[j, 16 * r + 15, pl.ds(0, 16)])
                ob_v[j, r, pl.ds(16, 16)] = (
                    acc_hi / s + rows_v[j, 16 * r + 15, pl.ds(16, 16)])
            pltpu.async_copy(ob_v.at[j],
                             out_hbm.at[pl.ds(wid * per_w + 8 * c, 8)],
                             osems[j])

        def wait_gather(j):
            pltpu.make_async_copy(table_hbm.at[idx_v.at[j]], rows_v.at[j],
                                  gsems[j]).wait()

        def wait_out(j):
            pltpu.make_async_copy(ob_v.at[j], out_hbm.at[pl.ds(0, 8)],
                                  osems[j]).wait()

        czero = jnp.minimum(wid * 0, 0)
        for j in range(3):
            build_fire(czero + j, j)

        def group_body(g, carry):
            for j in range(4):
                c = g * 4 + j
                build_fire(c + 3, (j + 3) % 4)
                wait_gather(j)

                @pl.when(g > 0)
                def _():
                    wait_out(j)

                compute(c, j)
            return carry

        lax.fori_loop(0, chunks // 4, group_body, 0)
        for j in range(3):
            wait_gather(j)
        for j in range(4):
            wait_out(j)

    return k


def _assemble_body(ab_ref, tmp_ref, virt_ref, out_ref):
    ab2 = ab_ref[0] * 2.0
    tt = jnp.transpose(tmp_ref[0], (1, 0)).reshape(H, 128, 128)
    vw = virt_ref[...].reshape(H, 1, 1)
    out_ref[0, :, 0:1, :] = ab2[None, 0:1, :] + vw
    out_ref[0, :, 1:, 0:1] = ab2[None, 1:, 0:1] + vw
    out_ref[0, :, 1:, 1:] = tt + ab2[None, 1:, 1:]


def _assemble(attn_bias, tmp, virt_w, B, N):
    return pl.pallas_call(
        _assemble_body,
        grid=(B,),
        in_specs=[
            pl.BlockSpec((1, N + 1, N + 1), lambda i: (i, 0, 0)),
            pl.BlockSpec((1, N * N, H), lambda i: (i, 0, 0)),
            pl.BlockSpec((1, H), lambda i: (0, 0)),
        ],
        out_specs=pl.BlockSpec((1, H, N + 1, N + 1), lambda i: (i, 0, 0, 0)),
        out_shape=jax.ShapeDtypeStruct((B, H, N + 1, N + 1), jnp.float32),
    )(attn_bias, tmp.reshape(B, N * N, H), virt_w)


def kernel(attn_bias, spatial_pos, x, edge_input, attn_edge_type,
           edge_enc_w, edge_dis_w, spatial_enc_w, virt_w):
    B, N = x.shape[0], x.shape[1]
    n_pairs = B * N * N
    table = _build_table(edge_enc_w, edge_dis_w, spatial_enc_w)
    edge_flat = jnp.transpose(edge_input[..., :MAX_DIST, :],
                              (0, 3, 4, 1, 2)).reshape(-1).astype(jnp.int32)
    spat_flat = spatial_pos.reshape(-1).astype(jnp.int32)
    tmp = _sc_gather_fn(n_pairs)(table, edge_flat, spat_flat)
    return _assemble(attn_bias, tmp, virt_w, B, N)
